# Initial kernel scaffold; baseline (speedup 1.0000x reference)
import jax
import jax.numpy as jnp
from jax.experimental import pallas as pl


def kernel(x, edge_index, W1, b1, W2, b2):
    raise NotImplementedError("write your pallas kernel here")


# speedup vs baseline: 31.3452x; 31.3452x over previous
import functools

import jax
import jax.numpy as jnp
from jax import lax
from jax.experimental import pallas as pl
from jax.experimental.pallas import tpu as pltpu
from jax.experimental.pallas import tpu_sc as plsc

N = 10000
E = 320000
D_IN = 128
D_HID = 16
D_OUT = 128

NP_ = 10240
NC, NS = 2, 16
NW = NC * NS
CB = 80
NCHUNK = E // (NW * CB)
ROWS_PER_TILE = NP_ // NS


def _mesh():
    return plsc.VectorSubcoreMesh(core_axis_name="c", subcore_axis_name="s")


@functools.partial(
    pl.kernel,
    out_type=jax.ShapeDtypeStruct((NC, NP_), jnp.float32),
    mesh=_mesh(),
    compiler_params=pltpu.CompilerParams(use_tc_tiling_on_sc=False),
    scratch_types=[
        pltpu.VMEM((NCHUNK, CB), jnp.int32),
        pltpu.VMEM((CB,), jnp.float32),
        pltpu.VMEM((CB,), jnp.float32),
        pltpu.VMEM_SHARED((NP_,), jnp.float32),
        pltpu.SemaphoreType.DMA,
    ],
)
def _deg_kernel(dst_hbm, out_hbm, idx_v, ones_v, zeros_v, acc_sh, sem):
    c = lax.axis_index("c")
    s = lax.axis_index("s")
    wid = s * NC + c

    for i in range(CB // 16):
        ones_v[pl.ds(i * 16, 16)] = jnp.ones((16,), jnp.float32)
        zeros_v[pl.ds(i * 16, 16)] = jnp.zeros((16,), jnp.float32)
    for i in range(ROWS_PER_TILE // CB):
        pltpu.sync_copy(
            zeros_v, acc_sh.at[pl.ds(s * ROWS_PER_TILE + i * CB, CB)])
    plsc.subcore_barrier()

    pltpu.sync_copy(dst_hbm.at[wid], idx_v)

    def body(j, carry):
        pltpu.sync_copy(ones_v, acc_sh.at[idx_v.at[j]], add=True)
        return carry

    lax.fori_loop(0, NCHUNK, body, 0)
    plsc.subcore_barrier()
    pltpu.sync_copy(acc_sh.at[pl.ds(s * ROWS_PER_TILE, ROWS_PER_TILE)],
                    out_hbm.at[c, pl.ds(s * ROWS_PER_TILE, ROWS_PER_TILE)])


@functools.partial(
    pl.kernel,
    out_type=jax.ShapeDtypeStruct((NC, NP_, D_HID), jnp.float32),
    mesh=_mesh(),
    compiler_params=pltpu.CompilerParams(use_tc_tiling_on_sc=False),
    scratch_types=[
        pltpu.VMEM((NCHUNK, CB), jnp.int32),
        pltpu.VMEM((NCHUNK, CB), jnp.int32),
        pltpu.VMEM((CB, D_HID), jnp.float32),
        pltpu.VMEM((CB, D_HID), jnp.float32),
        pltpu.VMEM_SHARED((NP_, D_HID), jnp.float32),
        pltpu.SemaphoreType.DMA,
    ],
)
def _agg_kernel(tab_hbm, src_hbm, dst_hbm, out_hbm,
                src_v, dst_v, rows_v, zero_v, acc_sh, sem):
    c = lax.axis_index("c")
    s = lax.axis_index("s")
    wid = s * NC + c

    for i in range(CB):
        zero_v[i, :] = jnp.zeros((D_HID,), jnp.float32)
    for i in range(ROWS_PER_TILE // CB):
        pltpu.sync_copy(
            zero_v, acc_sh.at[pl.ds(s * ROWS_PER_TILE + i * CB, CB)])
    plsc.subcore_barrier()

    pltpu.sync_copy(src_hbm.at[wid], src_v)
    pltpu.sync_copy(dst_hbm.at[wid], dst_v)

    def body(j, carry):
        pltpu.async_copy(tab_hbm.at[src_v.at[j]], rows_v, sem).wait()
        pltpu.sync_copy(rows_v, acc_sh.at[dst_v.at[j]], add=True)
        return carry

    lax.fori_loop(0, NCHUNK, body, 0)
    plsc.subcore_barrier()
    pltpu.sync_copy(acc_sh.at[pl.ds(s * ROWS_PER_TILE, ROWS_PER_TILE)],
                    out_hbm.at[c, pl.ds(s * ROWS_PER_TILE, ROWS_PER_TILE)])


def _k1_body(deg_ref, x_ref, w1_ref, xws_ref, dis_ref):
    deg = deg_ref[0, :] + deg_ref[1, :] + 1.0
    dis = lax.rsqrt(deg)[:, None]
    xw = jnp.dot(x_ref[...], w1_ref[...], preferred_element_type=jnp.float32)
    xws_ref[...] = xw * dis
    dis_ref[...] = dis


def _k3_body(p_ref, xws_ref, dis_ref, b1_ref, out_ref):
    acc = p_ref[0] + p_ref[1] + xws_ref[...]
    dis = dis_ref[...]
    h = jnp.maximum(acc * dis + b1_ref[...][None, :], 0.0)
    out_ref[...] = h * dis


def _k5_body(q_ref, hs_ref, dis_ref, w2_ref, b2_ref, out_ref):
    z = (q_ref[0] + q_ref[1] + hs_ref[...]) * dis_ref[...]
    out_ref[...] = (
        jnp.dot(z[:N], w2_ref[...], preferred_element_type=jnp.float32)
        + b2_ref[...][None, :]
    )


def kernel(x, edge_index, W1, b1, W2, b2):
    src = edge_index[0].reshape(NW, NCHUNK, CB)
    dst = edge_index[1].reshape(NW, NCHUNK, CB)
    x_pad = jnp.pad(x, ((0, NP_ - N), (0, 0)))

    deg_p = _deg_kernel(dst)

    xw_s, dis = pl.pallas_call(
        _k1_body,
        out_shape=(
            jax.ShapeDtypeStruct((NP_, D_HID), jnp.float32),
            jax.ShapeDtypeStruct((NP_, 1), jnp.float32),
        ),
        in_specs=[pl.BlockSpec(memory_space=pltpu.VMEM)] * 3,
        out_specs=(pl.BlockSpec(memory_space=pltpu.VMEM),
                   pl.BlockSpec(memory_space=pltpu.VMEM)),
    )(deg_p, x_pad, W1)

    p = _agg_kernel(xw_s, src, dst)

    h_s = pl.pallas_call(
        _k3_body,
        out_shape=jax.ShapeDtypeStruct((NP_, D_HID), jnp.float32),
        in_specs=[pl.BlockSpec(memory_space=pltpu.VMEM)] * 4,
        out_specs=pl.BlockSpec(memory_space=pltpu.VMEM),
    )(p, xw_s, dis, b1)

    q = _agg_kernel(h_s, src, dst)

    out = pl.pallas_call(
        _k5_body,
        out_shape=jax.ShapeDtypeStruct((N, D_OUT), jnp.float32),
        in_specs=[pl.BlockSpec(memory_space=pltpu.VMEM)] * 5,
        out_specs=pl.BlockSpec(memory_space=pltpu.VMEM),
    )(q, h_s, dis, W2, b2)

    return out


# speedup vs baseline: 42.0475x; 1.3414x over previous
import functools

import jax
import jax.numpy as jnp
from jax import lax
from jax.experimental import pallas as pl
from jax.experimental.pallas import tpu as pltpu
from jax.experimental.pallas import tpu_sc as plsc

N = 10000
E = 320000
D_IN = 128
D_HID = 16
D_OUT = 128

NP_ = 10240
NC, NS = 2, 16
NW = NC * NS
CB = 128
E_PAD = 327680
NCHUNK = E_PAD // (NW * CB)
ROWS_PER_TILE = NP_ // NS


def _mesh():
    return plsc.VectorSubcoreMesh(core_axis_name="c", subcore_axis_name="s")


@functools.partial(
    pl.kernel,
    out_type=jax.ShapeDtypeStruct((NC, NP_), jnp.float32),
    mesh=_mesh(),
    compiler_params=pltpu.CompilerParams(use_tc_tiling_on_sc=False),
    scratch_types=[
        pltpu.VMEM((NCHUNK, CB), jnp.int32),
        pltpu.VMEM((CB,), jnp.float32),
        pltpu.VMEM((CB,), jnp.float32),
        pltpu.VMEM_SHARED((NP_,), jnp.float32),
        pltpu.SemaphoreType.DMA,
    ],
)
def _deg_kernel(dst_hbm, out_hbm, idx_v, ones_v, zeros_v, acc_sh, sem):
    c = lax.axis_index("c")
    s = lax.axis_index("s")
    wid = s * NC + c

    for i in range(CB // 16):
        ones_v[pl.ds(i * 16, 16)] = jnp.ones((16,), jnp.float32)
        zeros_v[pl.ds(i * 16, 16)] = jnp.zeros((16,), jnp.float32)
    for i in range(ROWS_PER_TILE // CB):
        pltpu.sync_copy(
            zeros_v, acc_sh.at[pl.ds(s * ROWS_PER_TILE + i * CB, CB)])
    plsc.subcore_barrier()

    pltpu.sync_copy(dst_hbm.at[wid], idx_v)

    def body(j, carry):
        pltpu.sync_copy(ones_v, acc_sh.at[idx_v.at[j]], add=True)
        return carry

    lax.fori_loop(0, NCHUNK, body, 0)
    plsc.subcore_barrier()
    pltpu.sync_copy(acc_sh.at[pl.ds(s * ROWS_PER_TILE, ROWS_PER_TILE)],
                    out_hbm.at[c, pl.ds(s * ROWS_PER_TILE, ROWS_PER_TILE)])


@functools.partial(
    pl.kernel,
    out_type=jax.ShapeDtypeStruct((NC, NP_, D_HID), jnp.float32),
    mesh=_mesh(),
    compiler_params=pltpu.CompilerParams(use_tc_tiling_on_sc=False),
    scratch_types=[
        pltpu.VMEM((NCHUNK, CB), jnp.int32),
        pltpu.VMEM((NCHUNK, CB), jnp.int32),
        pltpu.VMEM((CB, D_HID), jnp.float32),
        pltpu.VMEM((CB, D_HID), jnp.float32),
        pltpu.VMEM((CB, D_HID), jnp.float32),
        pltpu.VMEM_SHARED((NP_, D_HID), jnp.float32),
        pltpu.SemaphoreType.DMA,
        pltpu.SemaphoreType.DMA,
    ],
)
def _agg_kernel(tab_hbm, src_hbm, dst_hbm, out_hbm,
                src_v, dst_v, rows0_v, rows1_v, zero_v, acc_sh, sem0, sem1):
    c = lax.axis_index("c")
    s = lax.axis_index("s")
    wid = s * NC + c
    rows = (rows0_v, rows1_v)
    sems = (sem0, sem1)

    for i in range(CB):
        zero_v[i, :] = jnp.zeros((D_HID,), jnp.float32)
    for i in range(ROWS_PER_TILE // CB):
        pltpu.sync_copy(
            zero_v, acc_sh.at[pl.ds(s * ROWS_PER_TILE + i * CB, CB)])
    plsc.subcore_barrier()

    pltpu.sync_copy(src_hbm.at[wid], src_v)
    pltpu.sync_copy(dst_hbm.at[wid], dst_v)

    pltpu.async_copy(tab_hbm.at[src_v.at[0]], rows[0], sems[0])

    def body(g, carry):
        for b in range(2):
            j = g * 2 + b
            pltpu.make_async_copy(
                tab_hbm.at[pl.ds(0, CB)], rows[b], sems[b]).wait()

            @pl.when(j + 1 < NCHUNK)
            def _():
                pltpu.async_copy(
                    tab_hbm.at[src_v.at[j + 1]], rows[1 - b], sems[1 - b])

            pltpu.sync_copy(rows[b], acc_sh.at[dst_v.at[j]], add=True)
        return carry

    lax.fori_loop(0, NCHUNK // 2, body, 0)
    plsc.subcore_barrier()
    pltpu.sync_copy(acc_sh.at[pl.ds(s * ROWS_PER_TILE, ROWS_PER_TILE)],
                    out_hbm.at[c, pl.ds(s * ROWS_PER_TILE, ROWS_PER_TILE)])


def _k1_body(deg_ref, x_ref, w1_ref, xws_ref, dis_ref):
    deg = deg_ref[0, :] + deg_ref[1, :] + 1.0
    dis = lax.rsqrt(deg)[:, None]
    xw = jnp.dot(x_ref[...], w1_ref[...], preferred_element_type=jnp.float32)
    xws_ref[...] = xw * dis
    dis_ref[...] = dis


def _k3_body(p_ref, xws_ref, dis_ref, b1_ref, out_ref):
    acc = p_ref[0] + p_ref[1] + xws_ref[...]
    dis = dis_ref[...]
    h = jnp.maximum(acc * dis + b1_ref[...][None, :], 0.0)
    out_ref[...] = h * dis


def _k5_body(q_ref, hs_ref, dis_ref, w2_ref, b2_ref, out_ref):
    z = (q_ref[0] + q_ref[1] + hs_ref[...]) * dis_ref[...]
    out_ref[...] = (
        jnp.dot(z[:N], w2_ref[...], preferred_element_type=jnp.float32)
        + b2_ref[...][None, :]
    )


def kernel(x, edge_index, W1, b1, W2, b2):
    pad_idx = (jnp.arange(E_PAD - E, dtype=jnp.int32) % (NP_ - N)) + N
    src = jnp.concatenate([edge_index[0], pad_idx]).reshape(NW, NCHUNK, CB)
    dst = jnp.concatenate([edge_index[1], pad_idx]).reshape(NW, NCHUNK, CB)
    x_pad = jnp.pad(x, ((0, NP_ - N), (0, 0)))

    deg_p = _deg_kernel(dst)

    xw_s, dis = pl.pallas_call(
        _k1_body,
        out_shape=(
            jax.ShapeDtypeStruct((NP_, D_HID), jnp.float32),
            jax.ShapeDtypeStruct((NP_, 1), jnp.float32),
        ),
        in_specs=[pl.BlockSpec(memory_space=pltpu.VMEM)] * 3,
        out_specs=(pl.BlockSpec(memory_space=pltpu.VMEM),
                   pl.BlockSpec(memory_space=pltpu.VMEM)),
    )(deg_p, x_pad, W1)

    p = _agg_kernel(xw_s, src, dst)

    h_s = pl.pallas_call(
        _k3_body,
        out_shape=jax.ShapeDtypeStruct((NP_, D_HID), jnp.float32),
        in_specs=[pl.BlockSpec(memory_space=pltpu.VMEM)] * 4,
        out_specs=pl.BlockSpec(memory_space=pltpu.VMEM),
    )(p, xw_s, dis, b1)

    q = _agg_kernel(h_s, src, dst)

    out = pl.pallas_call(
        _k5_body,
        out_shape=jax.ShapeDtypeStruct((N, D_OUT), jnp.float32),
        in_specs=[pl.BlockSpec(memory_space=pltpu.VMEM)] * 5,
        out_specs=pl.BlockSpec(memory_space=pltpu.VMEM),
    )(q, h_s, dis, W2, b2)

    return out


# speedup vs baseline: 67.6201x; 1.6082x over previous
import functools

import jax
import jax.numpy as jnp
from jax import lax
from jax.experimental import pallas as pl
from jax.experimental.pallas import tpu as pltpu
from jax.experimental.pallas import tpu_sc as plsc

N = 10000
E = 320000
D_IN = 128
D_HID = 16
D_OUT = 128

NP_ = 10240
NC, NS = 2, 16
NW = NC * NS
CB = 128
E_PAD = 327680
NCHUNK = E_PAD // (NW * CB)
ROWS_PER_TILE = NP_ // NS


def _mesh():
    return plsc.VectorSubcoreMesh(core_axis_name="c", subcore_axis_name="s")


@functools.partial(
    pl.kernel,
    out_type=jax.ShapeDtypeStruct((NC, NP_), jnp.float32),
    mesh=_mesh(),
    compiler_params=pltpu.CompilerParams(use_tc_tiling_on_sc=False),
    scratch_types=[
        pltpu.VMEM((NCHUNK, CB), jnp.int32),
        pltpu.VMEM((CB,), jnp.float32),
        pltpu.VMEM((CB,), jnp.float32),
        pltpu.VMEM_SHARED((NP_,), jnp.float32),
        pltpu.SemaphoreType.DMA,
    ],
)
def _deg_kernel(dst_hbm, out_hbm, idx_v, ones_v, zeros_v, acc_sh, sem):
    c = lax.axis_index("c")
    s = lax.axis_index("s")
    wid = s * NC + c

    for i in range(CB // 16):
        ones_v[pl.ds(i * 16, 16)] = jnp.ones((16,), jnp.float32)
        zeros_v[pl.ds(i * 16, 16)] = jnp.zeros((16,), jnp.float32)
    for i in range(ROWS_PER_TILE // CB):
        pltpu.sync_copy(
            zeros_v, acc_sh.at[pl.ds(s * ROWS_PER_TILE + i * CB, CB)])
    plsc.subcore_barrier()

    pltpu.sync_copy(dst_hbm.at[wid], idx_v)

    def body(j, carry):
        pltpu.async_copy(ones_v, acc_sh.at[idx_v.at[j]], sem, add=True)
        return carry

    lax.fori_loop(0, NCHUNK, body, 0)
    pltpu.make_async_copy(dst_hbm.at[wid], idx_v, sem).wait()
    plsc.subcore_barrier()
    pltpu.sync_copy(acc_sh.at[pl.ds(s * ROWS_PER_TILE, ROWS_PER_TILE)],
                    out_hbm.at[c, pl.ds(s * ROWS_PER_TILE, ROWS_PER_TILE)])


@functools.partial(
    pl.kernel,
    out_type=jax.ShapeDtypeStruct((NC, NP_, D_HID), jnp.float32),
    mesh=_mesh(),
    compiler_params=pltpu.CompilerParams(use_tc_tiling_on_sc=False),
    scratch_types=[
        pltpu.VMEM((NCHUNK, CB), jnp.int32),
        pltpu.VMEM((NCHUNK, CB), jnp.int32),
        [pltpu.VMEM((CB, D_HID), jnp.float32)] * 8,
        pltpu.VMEM((CB, D_HID), jnp.float32),
        pltpu.VMEM_SHARED((NP_, D_HID), jnp.float32),
        [pltpu.SemaphoreType.DMA] * 8,
        [pltpu.SemaphoreType.DMA] * 8,
    ],
)
def _agg_kernel(tab_hbm, src_hbm, dst_hbm, out_hbm,
                src_v, dst_v, rows, zero_v, acc_sh, gsem, ssem):
    NSLOT, LA = 8, 4
    c = lax.axis_index("c")
    s = lax.axis_index("s")
    wid = s * NC + c

    for i in range(CB):
        zero_v[i, :] = jnp.zeros((D_HID,), jnp.float32)
    for i in range(ROWS_PER_TILE // CB):
        pltpu.sync_copy(
            zero_v, acc_sh.at[pl.ds(s * ROWS_PER_TILE + i * CB, CB)])
    plsc.subcore_barrier()

    pltpu.sync_copy(src_hbm.at[wid], src_v)
    pltpu.sync_copy(dst_hbm.at[wid], dst_v)

    for j in range(LA):
        pltpu.async_copy(tab_hbm.at[src_v.at[j]], rows[j], gsem[j])

    def body(g, carry):
        for b in range(NSLOT):
            j = g * NSLOT + b
            b2 = (b + LA) % NSLOT
            pltpu.make_async_copy(
                tab_hbm.at[pl.ds(0, CB)], rows[b], gsem[b]).wait()
            pltpu.async_copy(rows[b], acc_sh.at[dst_v.at[j]], ssem[b],
                             add=True)
            @pl.when((j >= NSLOT - LA) & (j < NCHUNK - LA))
            def _():
                pltpu.make_async_copy(
                    tab_hbm.at[pl.ds(0, CB)], rows[b2], ssem[b2]).wait()

            @pl.when(j < NCHUNK - LA)
            def _():
                pltpu.async_copy(
                    tab_hbm.at[src_v.at[j + LA]], rows[b2], gsem[b2])
        return carry

    lax.fori_loop(0, NCHUNK // NSLOT, body, 0)
    for b in range(NSLOT):
        pltpu.make_async_copy(
            tab_hbm.at[pl.ds(0, CB)], rows[b], ssem[b]).wait()
    plsc.subcore_barrier()
    pltpu.sync_copy(acc_sh.at[pl.ds(s * ROWS_PER_TILE, ROWS_PER_TILE)],
                    out_hbm.at[c, pl.ds(s * ROWS_PER_TILE, ROWS_PER_TILE)])


def _k1_body(deg_ref, x_ref, w1_ref, xws_ref, dis_ref):
    deg = deg_ref[0, :] + deg_ref[1, :] + 1.0
    dis = lax.rsqrt(deg)[:, None]
    xw = jnp.dot(x_ref[...], w1_ref[...], preferred_element_type=jnp.float32)
    xws_ref[...] = xw * dis
    dis_ref[...] = dis


def _k3_body(p_ref, xws_ref, dis_ref, b1_ref, out_ref):
    acc = p_ref[0] + p_ref[1] + xws_ref[...]
    dis = dis_ref[...]
    h = jnp.maximum(acc * dis + b1_ref[...][None, :], 0.0)
    out_ref[...] = h * dis


def _k5_body(q_ref, hs_ref, dis_ref, w2_ref, b2_ref, out_ref):
    z = (q_ref[0] + q_ref[1] + hs_ref[...]) * dis_ref[...]
    out_ref[...] = (
        jnp.dot(z[:N], w2_ref[...], preferred_element_type=jnp.float32)
        + b2_ref[...][None, :]
    )


def kernel(x, edge_index, W1, b1, W2, b2):
    pad_idx = (jnp.arange(E_PAD - E, dtype=jnp.int32) % (NP_ - N)) + N
    src = jnp.concatenate([edge_index[0], pad_idx]).reshape(NW, NCHUNK, CB)
    dst = jnp.concatenate([edge_index[1], pad_idx]).reshape(NW, NCHUNK, CB)
    x_pad = jnp.pad(x, ((0, NP_ - N), (0, 0)))

    deg_p = _deg_kernel(dst)

    xw_s, dis = pl.pallas_call(
        _k1_body,
        out_shape=(
            jax.ShapeDtypeStruct((NP_, D_HID), jnp.float32),
            jax.ShapeDtypeStruct((NP_, 1), jnp.float32),
        ),
        in_specs=[pl.BlockSpec(memory_space=pltpu.VMEM)] * 3,
        out_specs=(pl.BlockSpec(memory_space=pltpu.VMEM),
                   pl.BlockSpec(memory_space=pltpu.VMEM)),
    )(deg_p, x_pad, W1)

    p = _agg_kernel(xw_s, src, dst)

    h_s = pl.pallas_call(
        _k3_body,
        out_shape=jax.ShapeDtypeStruct((NP_, D_HID), jnp.float32),
        in_specs=[pl.BlockSpec(memory_space=pltpu.VMEM)] * 4,
        out_specs=pl.BlockSpec(memory_space=pltpu.VMEM),
    )(p, xw_s, dis, b1)

    q = _agg_kernel(h_s, src, dst)

    out = pl.pallas_call(
        _k5_body,
        out_shape=jax.ShapeDtypeStruct((N, D_OUT), jnp.float32),
        in_specs=[pl.BlockSpec(memory_space=pltpu.VMEM)] * 5,
        out_specs=pl.BlockSpec(memory_space=pltpu.VMEM),
    )(q, h_s, dis, W2, b2)

    return out


# speedup vs baseline: 72.9339x; 1.0786x over previous
import functools

import jax
import jax.numpy as jnp
from jax import lax
from jax.experimental import pallas as pl
from jax.experimental.pallas import tpu as pltpu
from jax.experimental.pallas import tpu_sc as plsc

N = 10000
E = 320000
D_IN = 128
D_HID = 16
D_OUT = 128

NP_ = 10240
NC, NS = 2, 16
NW = NC * NS
CB = 128
E_PAD = 327680
NCHUNK = E_PAD // (NW * CB)
ROWS_PER_TILE = NP_ // NS


def _mesh():
    return plsc.VectorSubcoreMesh(core_axis_name="c", subcore_axis_name="s")


@functools.partial(
    pl.kernel,
    out_type=jax.ShapeDtypeStruct((NC, NP_), jnp.float32),
    mesh=_mesh(),
    compiler_params=pltpu.CompilerParams(use_tc_tiling_on_sc=False),
    scratch_types=[
        pltpu.VMEM((NCHUNK, CB), jnp.int32),
        pltpu.VMEM((CB,), jnp.float32),
        pltpu.VMEM((CB,), jnp.float32),
        pltpu.VMEM_SHARED((NP_,), jnp.float32),
        pltpu.SemaphoreType.DMA,
    ],
)
def _deg_kernel(dst_hbm, out_hbm, idx_v, ones_v, zeros_v, acc_sh, sem):
    c = lax.axis_index("c")
    s = lax.axis_index("s")
    wid = s * NC + c

    for i in range(CB // 16):
        ones_v[pl.ds(i * 16, 16)] = jnp.ones((16,), jnp.float32)
        zeros_v[pl.ds(i * 16, 16)] = jnp.zeros((16,), jnp.float32)
    for i in range(ROWS_PER_TILE // CB):
        pltpu.sync_copy(
            zeros_v, acc_sh.at[pl.ds(s * ROWS_PER_TILE + i * CB, CB)])
    plsc.subcore_barrier()

    pltpu.sync_copy(dst_hbm.at[wid], idx_v)

    def body(j, carry):
        pltpu.async_copy(ones_v, acc_sh.at[idx_v.at[j]], sem, add=True)
        return carry

    lax.fori_loop(0, NCHUNK, body, 0)
    pltpu.make_async_copy(dst_hbm.at[wid], idx_v, sem).wait()
    plsc.subcore_barrier()
    pltpu.sync_copy(acc_sh.at[pl.ds(s * ROWS_PER_TILE, ROWS_PER_TILE)],
                    out_hbm.at[c, pl.ds(s * ROWS_PER_TILE, ROWS_PER_TILE)])


@functools.partial(
    pl.kernel,
    out_type=jax.ShapeDtypeStruct((NC, NP_, D_HID), jnp.float32),
    mesh=_mesh(),
    compiler_params=pltpu.CompilerParams(use_tc_tiling_on_sc=False),
    scratch_types=[
        pltpu.VMEM((NCHUNK, CB), jnp.int32),
        pltpu.VMEM((NCHUNK, CB), jnp.int32),
        [pltpu.VMEM((CB, D_HID), jnp.float32)] * 8,
        pltpu.VMEM((CB, D_HID), jnp.float32),
        pltpu.VMEM_SHARED((NP_, D_HID), jnp.float32),
        pltpu.VMEM_SHARED((NP_, D_HID), jnp.float32),
        [pltpu.SemaphoreType.DMA] * 8,
        [pltpu.SemaphoreType.DMA] * 8,
    ],
)
def _agg_kernel(tab_hbm, src_hbm, dst_hbm, out_hbm,
                src_v, dst_v, rows, zero_v, acc_sh, tab_sh, gsem, ssem):
    NSLOT, LA = 8, 4
    c = lax.axis_index("c")
    s = lax.axis_index("s")
    wid = s * NC + c

    for i in range(CB):
        zero_v[i, :] = jnp.zeros((D_HID,), jnp.float32)
    for i in range(ROWS_PER_TILE // CB):
        pltpu.sync_copy(
            zero_v, acc_sh.at[pl.ds(s * ROWS_PER_TILE + i * CB, CB)])
    pltpu.sync_copy(tab_hbm.at[pl.ds(s * ROWS_PER_TILE, ROWS_PER_TILE)],
                    tab_sh.at[pl.ds(s * ROWS_PER_TILE, ROWS_PER_TILE)])
    plsc.subcore_barrier()

    pltpu.sync_copy(src_hbm.at[wid], src_v)
    pltpu.sync_copy(dst_hbm.at[wid], dst_v)

    for j in range(LA):
        pltpu.async_copy(tab_sh.at[src_v.at[j]], rows[j], gsem[j])

    def body(g, carry):
        for b in range(NSLOT):
            j = g * NSLOT + b
            b2 = (b + LA) % NSLOT
            pltpu.make_async_copy(
                tab_hbm.at[pl.ds(0, CB)], rows[b], gsem[b]).wait()
            pltpu.async_copy(rows[b], acc_sh.at[dst_v.at[j]], ssem[b],
                             add=True)
            @pl.when((j >= NSLOT - LA) & (j < NCHUNK - LA))
            def _():
                pltpu.make_async_copy(
                    tab_hbm.at[pl.ds(0, CB)], rows[b2], ssem[b2]).wait()

            @pl.when(j < NCHUNK - LA)
            def _():
                pltpu.async_copy(
                    tab_sh.at[src_v.at[j + LA]], rows[b2], gsem[b2])
        return carry

    lax.fori_loop(0, NCHUNK // NSLOT, body, 0)
    for b in range(NSLOT):
        pltpu.make_async_copy(
            tab_hbm.at[pl.ds(0, CB)], rows[b], ssem[b]).wait()
    plsc.subcore_barrier()
    pltpu.sync_copy(acc_sh.at[pl.ds(s * ROWS_PER_TILE, ROWS_PER_TILE)],
                    out_hbm.at[c, pl.ds(s * ROWS_PER_TILE, ROWS_PER_TILE)])


def _k1_body(deg_ref, x_ref, w1_ref, xws_ref, dis_ref):
    deg = deg_ref[0, :] + deg_ref[1, :] + 1.0
    dis = lax.rsqrt(deg)[:, None]
    xw = jnp.dot(x_ref[...], w1_ref[...], preferred_element_type=jnp.float32)
    xws_ref[...] = xw * dis
    dis_ref[...] = dis


def _k3_body(p_ref, xws_ref, dis_ref, b1_ref, out_ref):
    acc = p_ref[0] + p_ref[1] + xws_ref[...]
    dis = dis_ref[...]
    h = jnp.maximum(acc * dis + b1_ref[...][None, :], 0.0)
    out_ref[...] = h * dis


def _k5_body(q_ref, hs_ref, dis_ref, w2_ref, b2_ref, out_ref):
    z = (q_ref[0] + q_ref[1] + hs_ref[...]) * dis_ref[...]
    out_ref[...] = (
        jnp.dot(z[:N], w2_ref[...], preferred_element_type=jnp.float32)
        + b2_ref[...][None, :]
    )


def kernel(x, edge_index, W1, b1, W2, b2):
    pad_idx = (jnp.arange(E_PAD - E, dtype=jnp.int32) % (NP_ - N)) + N
    src = jnp.concatenate([edge_index[0], pad_idx]).reshape(NW, NCHUNK, CB)
    dst = jnp.concatenate([edge_index[1], pad_idx]).reshape(NW, NCHUNK, CB)
    x_pad = jnp.pad(x, ((0, NP_ - N), (0, 0)))

    deg_p = _deg_kernel(dst)

    xw_s, dis = pl.pallas_call(
        _k1_body,
        out_shape=(
            jax.ShapeDtypeStruct((NP_, D_HID), jnp.float32),
            jax.ShapeDtypeStruct((NP_, 1), jnp.float32),
        ),
        in_specs=[pl.BlockSpec(memory_space=pltpu.VMEM)] * 3,
        out_specs=(pl.BlockSpec(memory_space=pltpu.VMEM),
                   pl.BlockSpec(memory_space=pltpu.VMEM)),
    )(deg_p, x_pad, W1)

    p = _agg_kernel(xw_s, src, dst)

    h_s = pl.pallas_call(
        _k3_body,
        out_shape=jax.ShapeDtypeStruct((NP_, D_HID), jnp.float32),
        in_specs=[pl.BlockSpec(memory_space=pltpu.VMEM)] * 4,
        out_specs=pl.BlockSpec(memory_space=pltpu.VMEM),
    )(p, xw_s, dis, b1)

    q = _agg_kernel(h_s, src, dst)

    out = pl.pallas_call(
        _k5_body,
        out_shape=jax.ShapeDtypeStruct((N, D_OUT), jnp.float32),
        in_specs=[pl.BlockSpec(memory_space=pltpu.VMEM)] * 5,
        out_specs=pl.BlockSpec(memory_space=pltpu.VMEM),
    )(q, h_s, dis, W2, b2)

    return out


# speedup vs baseline: 77.8421x; 1.0673x over previous
import functools

import jax
import jax.numpy as jnp
from jax import lax
from jax.experimental import pallas as pl
from jax.experimental.pallas import tpu as pltpu
from jax.experimental.pallas import tpu_sc as plsc

N = 10000
E = 320000
D_IN = 128
D_HID = 16
D_OUT = 128

NP_ = 10240
NC, NS = 2, 16
NW = NC * NS
CB = 128
E_PAD = 327680
NCHUNK = E_PAD // (NW * CB)
ROWS_PER_TILE = NP_ // NS


def _mesh():
    return plsc.VectorSubcoreMesh(core_axis_name="c", subcore_axis_name="s")


@functools.partial(
    pl.kernel,
    out_type=jax.ShapeDtypeStruct((NC, NP_), jnp.float32),
    mesh=_mesh(),
    compiler_params=pltpu.CompilerParams(use_tc_tiling_on_sc=False),
    scratch_types=[
        pltpu.VMEM((NCHUNK, CB), jnp.int32),
        pltpu.VMEM((CB,), jnp.float32),
        pltpu.VMEM((CB,), jnp.float32),
        pltpu.VMEM_SHARED((NP_,), jnp.float32),
        pltpu.SemaphoreType.DMA,
    ],
)
def _deg_kernel(dst_hbm, out_hbm, idx_v, ones_v, zeros_v, acc_sh, sem):
    c = lax.axis_index("c")
    s = lax.axis_index("s")
    wid = s * NC + c

    for i in range(CB // 16):
        ones_v[pl.ds(i * 16, 16)] = jnp.ones((16,), jnp.float32)
        zeros_v[pl.ds(i * 16, 16)] = jnp.zeros((16,), jnp.float32)
    for i in range(ROWS_PER_TILE // CB):
        pltpu.sync_copy(
            zeros_v, acc_sh.at[pl.ds(s * ROWS_PER_TILE + i * CB, CB)])
    plsc.subcore_barrier()

    pltpu.sync_copy(dst_hbm.at[wid], idx_v)

    def body(j, carry):
        pltpu.async_copy(ones_v, acc_sh.at[idx_v.at[j]], sem, add=True)
        return carry

    lax.fori_loop(0, NCHUNK, body, 0)
    pltpu.make_async_copy(dst_hbm.at[wid], idx_v, sem).wait()
    plsc.subcore_barrier()
    pltpu.sync_copy(acc_sh.at[pl.ds(s * ROWS_PER_TILE, ROWS_PER_TILE)],
                    out_hbm.at[c, pl.ds(s * ROWS_PER_TILE, ROWS_PER_TILE)])


@functools.partial(
    pl.kernel,
    out_type=jax.ShapeDtypeStruct((NC, NP_, D_HID), jnp.float32),
    mesh=_mesh(),
    compiler_params=pltpu.CompilerParams(use_tc_tiling_on_sc=False),
    scratch_types=[
        pltpu.VMEM((NCHUNK, CB), jnp.int32),
        pltpu.VMEM((NCHUNK, CB), jnp.int32),
        [pltpu.VMEM((CB, D_HID), jnp.float32)] * 8,
        pltpu.VMEM((CB, D_HID), jnp.float32),
        pltpu.VMEM_SHARED((NP_, D_HID), jnp.float32),
        pltpu.VMEM_SHARED((NP_, D_HID), jnp.float32),
        [pltpu.SemaphoreType.DMA] * 8,
        [pltpu.SemaphoreType.DMA] * 8,
    ],
)
def _agg_kernel(tab_hbm, src_hbm, dst_hbm, out_hbm,
                src_v, dst_v, rows, zero_v, acc_sh, tab_sh, gsem, ssem):
    NSLOT, LA = 8, 4
    c = lax.axis_index("c")
    s = lax.axis_index("s")
    wid = s * NC + c

    for i in range(CB):
        zero_v[i, :] = jnp.zeros((D_HID,), jnp.float32)
    for i in range(ROWS_PER_TILE // CB):
        pltpu.sync_copy(
            zero_v, acc_sh.at[pl.ds(s * ROWS_PER_TILE + i * CB, CB)])
    pltpu.sync_copy(tab_hbm.at[pl.ds(s * ROWS_PER_TILE, ROWS_PER_TILE)],
                    tab_sh.at[pl.ds(s * ROWS_PER_TILE, ROWS_PER_TILE)])
    plsc.subcore_barrier()

    pltpu.sync_copy(src_hbm.at[wid], src_v)
    pltpu.sync_copy(dst_hbm.at[wid], dst_v)

    for j in range(LA):
        pltpu.async_copy(tab_sh.at[src_v.at[j]], rows[j], gsem[j])

    def body(g, carry):
        for b in range(NSLOT):
            j = g * NSLOT + b
            b2 = (b + LA) % NSLOT
            pltpu.make_async_copy(
                tab_hbm.at[pl.ds(0, CB)], rows[b], gsem[b]).wait()
            pltpu.async_copy(rows[b], acc_sh.at[dst_v.at[j]], ssem[b],
                             add=True)
            @pl.when((j >= NSLOT - LA) & (j < NCHUNK - LA))
            def _():
                pltpu.make_async_copy(
                    tab_hbm.at[pl.ds(0, CB)], rows[b2], ssem[b2]).wait()

            @pl.when(j < NCHUNK - LA)
            def _():
                pltpu.async_copy(
                    tab_sh.at[src_v.at[j + LA]], rows[b2], gsem[b2])
        return carry

    lax.fori_loop(0, NCHUNK // NSLOT, body, 0)
    for b in range(NSLOT):
        pltpu.make_async_copy(
            tab_hbm.at[pl.ds(0, CB)], rows[b], ssem[b]).wait()
    plsc.subcore_barrier()
    pltpu.sync_copy(acc_sh.at[pl.ds(s * ROWS_PER_TILE, ROWS_PER_TILE)],
                    out_hbm.at[c, pl.ds(s * ROWS_PER_TILE, ROWS_PER_TILE)])


@functools.partial(
    pl.kernel,
    out_type=(jax.ShapeDtypeStruct((NC, NP_, D_HID), jnp.float32),
              jax.ShapeDtypeStruct((NP_, D_HID), jnp.float32)),
    mesh=_mesh(),
    compiler_params=pltpu.CompilerParams(use_tc_tiling_on_sc=False),
    scratch_types=[
        pltpu.VMEM((NCHUNK, CB), jnp.int32),
        pltpu.VMEM((NCHUNK, CB), jnp.int32),
        [pltpu.VMEM((CB, D_HID), jnp.float32)] * 8,
        pltpu.VMEM((CB, D_HID), jnp.float32),
        [pltpu.VMEM((ROWS_PER_TILE, D_HID), jnp.float32)] * 4,
        pltpu.VMEM((ROWS_PER_TILE, D_HID), jnp.float32),
        pltpu.VMEM((D_HID,), jnp.float32),
        pltpu.VMEM_SHARED((NP_, D_HID), jnp.float32),
        pltpu.VMEM_SHARED((NP_, D_HID), jnp.float32),
        [pltpu.SemaphoreType.DMA] * 8,
        [pltpu.SemaphoreType.DMA] * 8,
    ],
)
def _relu_agg_kernel(p_hbm, xws_hbm, dis16_hbm, b1_hbm, src_hbm, dst_hbm,
                     out_hbm, hs_hbm, src_v, dst_v, rows, zero_v, bufs,
                     hrow_v, b1_v, acc_sh, htab_sh, gsem, ssem):
    NSLOT, LA = 8, 4
    c = lax.axis_index("c")
    s = lax.axis_index("s")
    wid = s * NC + c
    row0 = s * ROWS_PER_TILE
    bp0, bp1, bxw, bdis = bufs

    for i in range(CB):
        zero_v[i, :] = jnp.zeros((D_HID,), jnp.float32)
    for i in range(ROWS_PER_TILE // CB):
        pltpu.sync_copy(zero_v, acc_sh.at[pl.ds(row0 + i * CB, CB)])

    pltpu.sync_copy(b1_hbm, b1_v)
    pltpu.sync_copy(p_hbm.at[0, pl.ds(row0, ROWS_PER_TILE)], bp0)
    pltpu.sync_copy(p_hbm.at[1, pl.ds(row0, ROWS_PER_TILE)], bp1)
    pltpu.sync_copy(xws_hbm.at[pl.ds(row0, ROWS_PER_TILE)], bxw)
    pltpu.sync_copy(dis16_hbm.at[pl.ds(row0, ROWS_PER_TILE)], bdis)
    b1r = b1_v[...]

    def rbody(r, carry):
        d = bdis[r, :]
        t = (bp0[r, :] + bp1[r, :] + bxw[r, :]) * d + b1r
        hrow_v[r, :] = jnp.maximum(t, 0.0) * d
        return carry

    lax.fori_loop(0, ROWS_PER_TILE, rbody, 0)
    pltpu.sync_copy(hrow_v, htab_sh.at[pl.ds(row0, ROWS_PER_TILE)])

    @pl.when(c == 0)
    def _():
        pltpu.sync_copy(hrow_v, hs_hbm.at[pl.ds(row0, ROWS_PER_TILE)])

    plsc.subcore_barrier()

    pltpu.sync_copy(src_hbm.at[wid], src_v)
    pltpu.sync_copy(dst_hbm.at[wid], dst_v)

    for j in range(LA):
        pltpu.async_copy(htab_sh.at[src_v.at[j]], rows[j], gsem[j])

    def body(g, carry):
        for b in range(NSLOT):
            j = g * NSLOT + b
            b2 = (b + LA) % NSLOT
            pltpu.make_async_copy(
                p_hbm.at[0, pl.ds(0, CB)], rows[b], gsem[b]).wait()
            pltpu.async_copy(rows[b], acc_sh.at[dst_v.at[j]], ssem[b],
                             add=True)

            @pl.when((j >= NSLOT - LA) & (j < NCHUNK - LA))
            def _():
                pltpu.make_async_copy(
                    p_hbm.at[0, pl.ds(0, CB)], rows[b2], ssem[b2]).wait()

            @pl.when(j < NCHUNK - LA)
            def _():
                pltpu.async_copy(
                    htab_sh.at[src_v.at[j + LA]], rows[b2], gsem[b2])
        return carry

    lax.fori_loop(0, NCHUNK // NSLOT, body, 0)
    for b in range(NSLOT):
        pltpu.make_async_copy(
            p_hbm.at[0, pl.ds(0, CB)], rows[b], ssem[b]).wait()
    plsc.subcore_barrier()
    pltpu.sync_copy(acc_sh.at[pl.ds(row0, ROWS_PER_TILE)],
                    out_hbm.at[c, pl.ds(row0, ROWS_PER_TILE)])


def _k1_body(deg_ref, x_ref, w1_ref, xws_ref, dis_ref, dis16_ref):
    deg = deg_ref[0, :] + deg_ref[1, :] + 1.0
    dis = lax.rsqrt(deg)[:, None]
    xw = jnp.dot(x_ref[...], w1_ref[...], preferred_element_type=jnp.float32)
    xws_ref[...] = xw * dis
    dis_ref[...] = dis
    dis16_ref[...] = jnp.broadcast_to(dis, (NP_, D_HID))


def _k5_body(q_ref, hs_ref, dis_ref, w2_ref, b2_ref, out_ref):
    z = (q_ref[0] + q_ref[1] + hs_ref[...]) * dis_ref[...]
    out_ref[...] = (
        jnp.dot(z[:N], w2_ref[...], preferred_element_type=jnp.float32)
        + b2_ref[...][None, :]
    )


def kernel(x, edge_index, W1, b1, W2, b2):
    pad_idx = (jnp.arange(E_PAD - E, dtype=jnp.int32) % (NP_ - N)) + N
    src = jnp.concatenate([edge_index[0], pad_idx]).reshape(NW, NCHUNK, CB)
    dst = jnp.concatenate([edge_index[1], pad_idx]).reshape(NW, NCHUNK, CB)
    x_pad = jnp.pad(x, ((0, NP_ - N), (0, 0)))

    deg_p = _deg_kernel(dst)

    xw_s, dis, dis16 = pl.pallas_call(
        _k1_body,
        out_shape=(
            jax.ShapeDtypeStruct((NP_, D_HID), jnp.float32),
            jax.ShapeDtypeStruct((NP_, 1), jnp.float32),
            jax.ShapeDtypeStruct((NP_, D_HID), jnp.float32),
        ),
        in_specs=[pl.BlockSpec(memory_space=pltpu.VMEM)] * 3,
        out_specs=(pl.BlockSpec(memory_space=pltpu.VMEM),) * 3,
    )(deg_p, x_pad, W1)

    p = _agg_kernel(xw_s, src, dst)

    q, h_s = _relu_agg_kernel(p, xw_s, dis16, b1, src, dst)

    out = pl.pallas_call(
        _k5_body,
        out_shape=jax.ShapeDtypeStruct((N, D_OUT), jnp.float32),
        in_specs=[pl.BlockSpec(memory_space=pltpu.VMEM)] * 5,
        out_specs=pl.BlockSpec(memory_space=pltpu.VMEM),
    )(q, h_s, dis, W2, b2)

    return out


# speedup vs baseline: 78.9080x; 1.0137x over previous
import functools

import jax
import jax.numpy as jnp
from jax import lax
from jax.experimental import pallas as pl
from jax.experimental.pallas import tpu as pltpu
from jax.experimental.pallas import tpu_sc as plsc

N = 10000
E = 320000
D_IN = 128
D_HID = 16
D_OUT = 128

NP_ = 10240
NC, NS = 2, 16
NW = NC * NS
CB = 128
E_PAD = 327680
NCHUNK = E_PAD // (NW * CB)
ROWS_PER_TILE = NP_ // NS


def _mesh():
    return plsc.VectorSubcoreMesh(core_axis_name="c", subcore_axis_name="s")


@functools.partial(
    pl.kernel,
    out_type=jax.ShapeDtypeStruct((NC, NP_), jnp.float32),
    mesh=_mesh(),
    compiler_params=pltpu.CompilerParams(use_tc_tiling_on_sc=False),
    scratch_types=[
        pltpu.VMEM((NCHUNK, CB), jnp.int32),
        pltpu.VMEM((CB,), jnp.float32),
        pltpu.VMEM((CB,), jnp.float32),
        pltpu.VMEM_SHARED((NP_,), jnp.float32),
        pltpu.SemaphoreType.DMA,
    ],
)
def _deg_kernel(dst_hbm, out_hbm, idx_v, ones_v, zeros_v, acc_sh, sem):
    c = lax.axis_index("c")
    s = lax.axis_index("s")
    wid = s * NC + c

    for i in range(CB // 16):
        ones_v[pl.ds(i * 16, 16)] = jnp.ones((16,), jnp.float32)
        zeros_v[pl.ds(i * 16, 16)] = jnp.zeros((16,), jnp.float32)
    for i in range(ROWS_PER_TILE // CB):
        pltpu.sync_copy(
            zeros_v, acc_sh.at[pl.ds(s * ROWS_PER_TILE + i * CB, CB)])
    plsc.subcore_barrier()

    pltpu.sync_copy(dst_hbm.at[wid], idx_v)

    def body(j, carry):
        pltpu.async_copy(ones_v, acc_sh.at[idx_v.at[j]], sem, add=True)
        return carry

    lax.fori_loop(0, NCHUNK, body, 0)
    pltpu.make_async_copy(dst_hbm.at[wid], idx_v, sem).wait()
    plsc.subcore_barrier()
    pltpu.sync_copy(acc_sh.at[pl.ds(s * ROWS_PER_TILE, ROWS_PER_TILE)],
                    out_hbm.at[c, pl.ds(s * ROWS_PER_TILE, ROWS_PER_TILE)])


@functools.partial(
    pl.kernel,
    out_type=jax.ShapeDtypeStruct((NC, NP_, D_HID), jnp.float32),
    mesh=_mesh(),
    compiler_params=pltpu.CompilerParams(use_tc_tiling_on_sc=False),
    scratch_types=[
        pltpu.VMEM((NCHUNK, CB), jnp.int32),
        pltpu.VMEM((NCHUNK, CB), jnp.int32),
        [pltpu.VMEM((CB, D_HID), jnp.float32)] * 8,
        pltpu.VMEM((CB, D_HID), jnp.float32),
        pltpu.VMEM_SHARED((NP_, D_HID), jnp.float32),
        pltpu.VMEM_SHARED((NP_, D_HID), jnp.float32),
        [pltpu.SemaphoreType.DMA] * 8,
        [pltpu.SemaphoreType.DMA] * 8,
    ],
)
def _agg_kernel(tab_hbm, src_hbm, dst_hbm, out_hbm,
                src_v, dst_v, rows, zero_v, acc_sh, tab_sh, gsem, ssem):
    NSLOT, LA = 8, 4
    c = lax.axis_index("c")
    s = lax.axis_index("s")
    wid = s * NC + c

    for i in range(CB):
        zero_v[i, :] = jnp.zeros((D_HID,), jnp.float32)
    for i in range(ROWS_PER_TILE // CB):
        pltpu.sync_copy(
            zero_v, acc_sh.at[pl.ds(s * ROWS_PER_TILE + i * CB, CB)])
    pltpu.sync_copy(tab_hbm.at[pl.ds(s * ROWS_PER_TILE, ROWS_PER_TILE)],
                    tab_sh.at[pl.ds(s * ROWS_PER_TILE, ROWS_PER_TILE)])
    plsc.subcore_barrier()

    pltpu.sync_copy(src_hbm.at[wid], src_v)
    pltpu.sync_copy(dst_hbm.at[wid], dst_v)

    for j in range(LA):
        pltpu.async_copy(tab_sh.at[src_v.at[j]], rows[j], gsem[j])

    def body(g, carry):
        for b in range(NSLOT):
            j = g * NSLOT + b
            b2 = (b + LA) % NSLOT
            pltpu.make_async_copy(
                tab_hbm.at[pl.ds(0, CB)], rows[b], gsem[b]).wait()
            pltpu.async_copy(rows[b], acc_sh.at[dst_v.at[j]], ssem[b],
                             add=True)
            @pl.when((j >= NSLOT - LA) & (j < NCHUNK - LA))
            def _():
                pltpu.make_async_copy(
                    tab_hbm.at[pl.ds(0, CB)], rows[b2], ssem[b2]).wait()

            @pl.when(j < NCHUNK - LA)
            def _():
                pltpu.async_copy(
                    tab_sh.at[src_v.at[j + LA]], rows[b2], gsem[b2])
        return carry

    lax.fori_loop(0, NCHUNK // NSLOT, body, 0)
    for b in range(NSLOT):
        pltpu.make_async_copy(
            tab_hbm.at[pl.ds(0, CB)], rows[b], ssem[b]).wait()
    plsc.subcore_barrier()
    pltpu.sync_copy(acc_sh.at[pl.ds(s * ROWS_PER_TILE, ROWS_PER_TILE)],
                    out_hbm.at[c, pl.ds(s * ROWS_PER_TILE, ROWS_PER_TILE)])


@functools.partial(
    pl.kernel,
    out_type=(jax.ShapeDtypeStruct((NC, NP_, D_HID), jnp.float32),
              jax.ShapeDtypeStruct((NP_, D_HID), jnp.float32)),
    mesh=_mesh(),
    compiler_params=pltpu.CompilerParams(use_tc_tiling_on_sc=False),
    scratch_types=[
        pltpu.VMEM((NCHUNK, CB), jnp.int32),
        pltpu.VMEM((NCHUNK, CB), jnp.int32),
        [pltpu.VMEM((CB, D_HID), jnp.float32)] * 8,
        pltpu.VMEM((CB, D_HID), jnp.float32),
        [pltpu.VMEM((ROWS_PER_TILE, D_HID), jnp.float32)] * 4,
        pltpu.VMEM((ROWS_PER_TILE, D_HID), jnp.float32),
        pltpu.VMEM((D_HID,), jnp.float32),
        pltpu.VMEM_SHARED((NP_, D_HID), jnp.float32),
        pltpu.VMEM_SHARED((NP_, D_HID), jnp.float32),
        [pltpu.SemaphoreType.DMA] * 8,
        [pltpu.SemaphoreType.DMA] * 8,
    ],
)
def _relu_agg_kernel(p_hbm, xws_hbm, dis16_hbm, b1_hbm, src_hbm, dst_hbm,
                     out_hbm, hs_hbm, src_v, dst_v, rows, zero_v, bufs,
                     hrow_v, b1_v, acc_sh, htab_sh, gsem, ssem):
    NSLOT, LA = 8, 4
    c = lax.axis_index("c")
    s = lax.axis_index("s")
    wid = s * NC + c
    row0 = s * ROWS_PER_TILE
    bp0, bp1, bxw, bdis = bufs

    for i in range(CB):
        zero_v[i, :] = jnp.zeros((D_HID,), jnp.float32)
    for i in range(ROWS_PER_TILE // CB):
        pltpu.sync_copy(zero_v, acc_sh.at[pl.ds(row0 + i * CB, CB)])

    pltpu.sync_copy(b1_hbm, b1_v)
    pltpu.sync_copy(p_hbm.at[0, pl.ds(row0, ROWS_PER_TILE)], bp0)
    pltpu.sync_copy(p_hbm.at[1, pl.ds(row0, ROWS_PER_TILE)], bp1)
    pltpu.sync_copy(xws_hbm.at[pl.ds(row0, ROWS_PER_TILE)], bxw)
    pltpu.sync_copy(dis16_hbm.at[pl.ds(row0, ROWS_PER_TILE)], bdis)
    b1r = b1_v[...]

    def rbody(r4, carry):
        for u in range(4):
            r = r4 * 4 + u
            d = bdis[r, :]
            t = (bp0[r, :] + bp1[r, :] + bxw[r, :]) * d + b1r
            hrow_v[r, :] = jnp.maximum(t, 0.0) * d
        return carry

    lax.fori_loop(0, ROWS_PER_TILE // 4, rbody, 0)
    pltpu.sync_copy(hrow_v, htab_sh.at[pl.ds(row0, ROWS_PER_TILE)])

    @pl.when(c == 0)
    def _():
        pltpu.sync_copy(hrow_v, hs_hbm.at[pl.ds(row0, ROWS_PER_TILE)])

    plsc.subcore_barrier()

    pltpu.sync_copy(src_hbm.at[wid], src_v)
    pltpu.sync_copy(dst_hbm.at[wid], dst_v)

    for j in range(LA):
        pltpu.async_copy(htab_sh.at[src_v.at[j]], rows[j], gsem[j])

    def body(g, carry):
        for b in range(NSLOT):
            j = g * NSLOT + b
            b2 = (b + LA) % NSLOT
            pltpu.make_async_copy(
                p_hbm.at[0, pl.ds(0, CB)], rows[b], gsem[b]).wait()
            pltpu.async_copy(rows[b], acc_sh.at[dst_v.at[j]], ssem[b],
                             add=True)

            @pl.when((j >= NSLOT - LA) & (j < NCHUNK - LA))
            def _():
                pltpu.make_async_copy(
                    p_hbm.at[0, pl.ds(0, CB)], rows[b2], ssem[b2]).wait()

            @pl.when(j < NCHUNK - LA)
            def _():
                pltpu.async_copy(
                    htab_sh.at[src_v.at[j + LA]], rows[b2], gsem[b2])
        return carry

    lax.fori_loop(0, NCHUNK // NSLOT, body, 0)
    for b in range(NSLOT):
        pltpu.make_async_copy(
            p_hbm.at[0, pl.ds(0, CB)], rows[b], ssem[b]).wait()
    plsc.subcore_barrier()
    pltpu.sync_copy(acc_sh.at[pl.ds(row0, ROWS_PER_TILE)],
                    out_hbm.at[c, pl.ds(row0, ROWS_PER_TILE)])


def _k1_body(deg_ref, x_ref, w1_ref, xws_ref, dis_ref, dis16_ref):
    deg = deg_ref[0, :] + deg_ref[1, :] + 1.0
    dis = lax.rsqrt(deg)[:, None]
    xw = jnp.dot(x_ref[...], w1_ref[...], preferred_element_type=jnp.float32)
    xws_ref[pl.ds(0, N), :] = xw * dis[:N]
    xws_ref[pl.ds(N, NP_ - N), :] = jnp.zeros((NP_ - N, D_HID), jnp.float32)
    dis_ref[...] = dis
    dis16_ref[...] = jnp.broadcast_to(dis, (NP_, D_HID))


def _k5_body(q_ref, hs_ref, dis_ref, w2_ref, b2_ref, out_ref):
    z = (q_ref[0] + q_ref[1] + hs_ref[...]) * dis_ref[...]
    out_ref[...] = (
        jnp.dot(z[:N], w2_ref[...], preferred_element_type=jnp.float32)
        + b2_ref[...][None, :]
    )


def kernel(x, edge_index, W1, b1, W2, b2):
    pad_idx = (jnp.arange(E_PAD - E, dtype=jnp.int32) % (NP_ - N)) + N
    src = jnp.concatenate([edge_index[0], pad_idx]).reshape(NW, NCHUNK, CB)
    dst = jnp.concatenate([edge_index[1], pad_idx]).reshape(NW, NCHUNK, CB)

    deg_p = _deg_kernel(dst)

    xw_s, dis, dis16 = pl.pallas_call(
        _k1_body,
        out_shape=(
            jax.ShapeDtypeStruct((NP_, D_HID), jnp.float32),
            jax.ShapeDtypeStruct((NP_, 1), jnp.float32),
            jax.ShapeDtypeStruct((NP_, D_HID), jnp.float32),
        ),
        in_specs=[pl.BlockSpec(memory_space=pltpu.VMEM)] * 3,
        out_specs=(pl.BlockSpec(memory_space=pltpu.VMEM),) * 3,
    )(deg_p, x, W1)

    p = _agg_kernel(xw_s, src, dst)

    q, h_s = _relu_agg_kernel(p, xw_s, dis16, b1, src, dst)

    out = pl.pallas_call(
        _k5_body,
        out_shape=jax.ShapeDtypeStruct((N, D_OUT), jnp.float32),
        in_specs=[pl.BlockSpec(memory_space=pltpu.VMEM)] * 5,
        out_specs=pl.BlockSpec(memory_space=pltpu.VMEM),
    )(q, h_s, dis, W2, b2)

    return out
